# baseline (speedup 1.0000x reference)
import functools

import jax
import jax.numpy as jnp
from jax import lax
from jax.experimental import pallas as pl
from jax.experimental.pallas import tpu as pltpu
from jax.experimental.pallas import tpu_sc as plsc

_THRESH = 0.5
_MAX_BOX = 100
_SCORE_THRESH = 0.05
_NEG = -1e9
_LOW = -3.0e38

_NS = 16
_L = 16
_NPAD = 20480
_PER = _NPAD // _NS
_SLICES = _PER // _L


def _sc_body(x1_h, y1_h, x2_h, y2_h, sc_h, out_h, ex_h,
             x1v, y1v, x2v, y2v, arv, swv, pubv, candv, outv):
    sid = lax.axis_index("s")
    base = sid * _PER
    li = lax.iota(jnp.int32, _L)
    lif = li.astype(jnp.float32)

    pltpu.sync_copy(x1_h.at[pl.ds(base, _PER)], x1v)
    pltpu.sync_copy(y1_h.at[pl.ds(base, _PER)], y1v)
    pltpu.sync_copy(x2_h.at[pl.ds(base, _PER)], x2v)
    pltpu.sync_copy(y2_h.at[pl.ds(base, _PER)], y2v)
    pltpu.sync_copy(sc_h.at[pl.ds(base, _PER)], swv)

    def init_slice(j, carry):
        vmax, vidx = carry
        sl = pl.ds(j * _L, _L)
        xs1 = x1v[sl]
        ys1 = y1v[sl]
        xs2 = x2v[sl]
        ys2 = y2v[sl]
        arv[sl] = (xs2 - xs1) * (ys2 - ys1)
        sws = swv[sl]
        gi = jnp.full((_L,), base + j * _L, jnp.int32) + li
        m = sws > vmax
        return jnp.where(m, sws, vmax), jnp.where(m, gi, vidx)

    carry0 = (jnp.full((_L,), _LOW, jnp.float32), jnp.zeros((_L,), jnp.int32))
    vmax, vidx = lax.fori_loop(0, _SLICES, init_slice, carry0)

    def round_fn(r, carry):
        vmax, vidx = carry
        sk, _ = plsc.sort_key_val(vmax, vidx, descending=True)
        mx = sk[0]
        cand_i = jnp.where(vmax == mx, vidx, jnp.int32(2147483647))
        si, _ = plsc.sort_key_val(cand_i, vidx)
        lidx = si[0]
        offv = jnp.full((_L,), lidx - base, jnp.int32)
        gx1 = plsc.load_gather(x1v, [offv])
        gy1 = plsc.load_gather(y1v, [offv])
        gx2 = plsc.load_gather(x2v, [offv])
        gy2 = plsc.load_gather(y2v, [offv])
        pub = jnp.where(li == 0, jnp.full((_L,), mx),
              jnp.where(li == 1, jnp.full((_L,), lidx.astype(jnp.float32)),
              jnp.where(li == 2, gx1,
              jnp.where(li == 3, gy1,
              jnp.where(li == 4, gx2, gy2)))))
        pubv[...] = pub
        pltpu.sync_copy(pubv, ex_h.at[sid])
        plsc.subcore_barrier()
        pltpu.sync_copy(ex_h, candv)
        plsc.subcore_barrier()

        def scan_c(j, c):
            bs, bj = c
            v = candv[j, :][0]
            better = v > bs
            return jnp.where(better, v, bs), jnp.where(better, j, bj)

        bs, bj = lax.fori_loop(0, _NS, scan_c,
                               (jnp.float32(_LOW), jnp.int32(0)))
        wrow = candv[bj, :]
        bidx = wrow[1].astype(jnp.int32)
        wx1 = wrow[2]
        wy1 = wrow[3]
        wx2 = wrow[4]
        wy2 = wrow[5]

        @pl.when(sid == 0)
        def _():
            outv[r, :] = wrow

        area_b = (wx2 - wx1) * (wy2 - wy1)
        wx1v = jnp.full((_L,), wx1)
        wy1v = jnp.full((_L,), wy1)
        wx2v = jnp.full((_L,), wx2)
        wy2v = jnp.full((_L,), wy2)
        abv = jnp.full((_L,), area_b)
        bidxv = jnp.full((_L,), bidx, jnp.int32)

        def dec_slice(j, c):
            nvmax, nvidx = c
            sl = pl.ds(j * _L, _L)
            sws = swv[sl]
            xs1 = x1v[sl]
            ys1 = y1v[sl]
            xs2 = x2v[sl]
            ys2 = y2v[sl]
            ars = arv[sl]
            iw = jnp.maximum(jnp.minimum(wx2v, xs2) - jnp.maximum(wx1v, xs1), 0.0)
            ih = jnp.maximum(jnp.minimum(wy2v, ys2) - jnp.maximum(wy1v, ys1), 0.0)
            inter = iw * ih
            iou = inter / jnp.maximum(abv + ars - inter, 1e-9)
            dec = jnp.where(iou > _THRESH, 1.0 - iou, 1.0)
            gi = jnp.full((_L,), base + j * _L, jnp.int32) + li
            nsw = jnp.where(gi == bidxv, _NEG, sws * dec)
            swv[sl] = nsw
            m = nsw > nvmax
            return jnp.where(m, nsw, nvmax), jnp.where(m, gi, nvidx)

        return lax.fori_loop(0, _SLICES, dec_slice, carry0)

    lax.fori_loop(0, _MAX_BOX, round_fn, (vmax, vidx))

    @pl.when(sid == 0)
    def _():
        pltpu.sync_copy(outv, out_h)


_sc_kernel = functools.partial(
    pl.kernel,
    mesh=plsc.VectorSubcoreMesh(core_axis_name="c", subcore_axis_name="s",
                                num_cores=1),
    compiler_params=pltpu.CompilerParams(needs_layout_passes=False),
    out_type=(jax.ShapeDtypeStruct((_MAX_BOX, _L), jnp.float32),
              jax.ShapeDtypeStruct((_NS, _L), jnp.float32)),
    scratch_types=[
        pltpu.VMEM((_PER,), jnp.float32),
        pltpu.VMEM((_PER,), jnp.float32),
        pltpu.VMEM((_PER,), jnp.float32),
        pltpu.VMEM((_PER,), jnp.float32),
        pltpu.VMEM((_PER,), jnp.float32),
        pltpu.VMEM((_PER,), jnp.float32),
        pltpu.VMEM((_L,), jnp.float32),
        pltpu.VMEM((_NS, _L), jnp.float32),
        pltpu.VMEM((_MAX_BOX, _L), jnp.float32),
    ],
)(_sc_body)


def kernel(boxes, scores):
    n = scores.shape[0]
    boxes = boxes.astype(jnp.float32)
    pad = _NPAD - n
    x1 = jnp.pad(boxes[:, 0], (0, pad))
    y1 = jnp.pad(boxes[:, 1], (0, pad))
    x2 = jnp.pad(boxes[:, 2], (0, pad))
    y2 = jnp.pad(boxes[:, 3], (0, pad))
    sc = jnp.pad(scores.astype(jnp.float32), (0, pad), constant_values=_NEG)
    out, _ = _sc_kernel(x1, y1, x2, y2, sc)
    s = out[:, 0]
    valid = (s >= _SCORE_THRESH).astype(boxes.dtype)
    kept_boxes = out[:, 2:6] * valid[:, None]
    kept_scores = s * valid
    return jnp.concatenate([kept_boxes, kept_scores[:, None]], axis=1)


# speedup vs baseline: 1.0101x; 1.0101x over previous
import functools

import jax
import jax.numpy as jnp
from jax import lax
from jax.experimental import pallas as pl
from jax.experimental.pallas import tpu as pltpu
from jax.experimental.pallas import tpu_sc as plsc

_THRESH = 0.5
_MAX_BOX = 100
_SCORE_THRESH = 0.05
_NEG = -1e9
_LOW = -3.0e38

_NS = 16
_L = 16
_NPAD = 20480
_PER = _NPAD // _NS
_SLICES = _PER // _L
_UNROLL = 4


def _sc_body(x1_h, y1_h, x2_h, y2_h, sc_h, out_h, ex_h,
             x1v, y1v, x2v, y2v, arv, swv, pubv, candv, outv):
    sid = lax.axis_index("s")
    base = sid * _PER
    li = lax.iota(jnp.int32, _L)

    pltpu.sync_copy(x1_h.at[pl.ds(base, _PER)], x1v)
    pltpu.sync_copy(y1_h.at[pl.ds(base, _PER)], y1v)
    pltpu.sync_copy(x2_h.at[pl.ds(base, _PER)], x2v)
    pltpu.sync_copy(y2_h.at[pl.ds(base, _PER)], y2v)
    pltpu.sync_copy(sc_h.at[pl.ds(base, _PER)], swv)

    def init_slice(j, carry):
        vmax, vidx = carry
        sl = pl.ds(j * _L, _L)
        xs1 = x1v[sl]
        ys1 = y1v[sl]
        xs2 = x2v[sl]
        ys2 = y2v[sl]
        arv[sl] = (xs2 - xs1) * (ys2 - ys1)
        sws = swv[sl]
        gi = jnp.full((_L,), base + j * _L, jnp.int32) + li
        m = sws > vmax
        return jnp.where(m, sws, vmax), jnp.where(m, gi, vidx)

    carry0 = (jnp.full((_L,), _LOW, jnp.float32), jnp.zeros((_L,), jnp.int32))
    vmax, vidx = lax.fori_loop(0, _SLICES, init_slice, carry0,
                               unroll=_UNROLL)

    def round_fn(r, carry):
        vmax, vidx = carry
        sk, _ = plsc.sort_key_val(vmax, vidx, descending=True)
        mx = sk[0]
        cand_i = jnp.where(vmax == mx, vidx, jnp.int32(2147483647))
        si, _ = plsc.sort_key_val(cand_i, vidx)
        lidx = si[0]
        offv = jnp.full((_L,), lidx - base, jnp.int32)
        gx1 = plsc.load_gather(x1v, [offv])
        gy1 = plsc.load_gather(y1v, [offv])
        gx2 = plsc.load_gather(x2v, [offv])
        gy2 = plsc.load_gather(y2v, [offv])
        pub = jnp.where(li == 0, jnp.full((_L,), mx),
              jnp.where(li == 1, jnp.full((_L,), lidx.astype(jnp.float32)),
              jnp.where(li == 2, gx1,
              jnp.where(li == 3, gy1,
              jnp.where(li == 4, gx2, gy2)))))
        pubv[...] = pub
        bank = lax.rem(r, 2)
        pltpu.sync_copy(pubv, ex_h.at[bank, sid])
        plsc.subcore_barrier()
        pltpu.sync_copy(ex_h.at[bank], candv)

        bs = jnp.float32(_LOW)
        wrow = candv[0, :]
        for j in range(_NS):
            row = candv[j, :]
            v = row[0]
            better = v > bs
            bs = jnp.where(better, v, bs)
            wrow = jnp.where(better, row, wrow)
        bidx = wrow[1].astype(jnp.int32)
        wx1 = wrow[2]
        wy1 = wrow[3]
        wx2 = wrow[4]
        wy2 = wrow[5]

        @pl.when(sid == 0)
        def _():
            outv[r, :] = wrow

        area_b = (wx2 - wx1) * (wy2 - wy1)
        wx1v = jnp.full((_L,), wx1)
        wy1v = jnp.full((_L,), wy1)
        wx2v = jnp.full((_L,), wx2)
        wy2v = jnp.full((_L,), wy2)
        abv = jnp.full((_L,), area_b)
        bidxv = jnp.full((_L,), bidx, jnp.int32)

        def dec_slice(j, c):
            nvmax, nvidx = c
            sl = pl.ds(j * _L, _L)
            sws = swv[sl]
            xs1 = x1v[sl]
            ys1 = y1v[sl]
            xs2 = x2v[sl]
            ys2 = y2v[sl]
            ars = arv[sl]
            iw = jnp.maximum(jnp.minimum(wx2v, xs2) - jnp.maximum(wx1v, xs1), 0.0)
            ih = jnp.maximum(jnp.minimum(wy2v, ys2) - jnp.maximum(wy1v, ys1), 0.0)
            inter = iw * ih
            iou = inter / jnp.maximum(abv + ars - inter, 1e-9)
            dec = jnp.where(iou > _THRESH, 1.0 - iou, 1.0)
            gi = jnp.full((_L,), base + j * _L, jnp.int32) + li
            nsw = jnp.where(gi == bidxv, _NEG, sws * dec)
            swv[sl] = nsw
            m = nsw > nvmax
            return jnp.where(m, nsw, nvmax), jnp.where(m, gi, nvidx)

        return lax.fori_loop(0, _SLICES, dec_slice, carry0, unroll=_UNROLL)

    lax.fori_loop(0, _MAX_BOX, round_fn, (vmax, vidx))

    @pl.when(sid == 0)
    def _():
        pltpu.sync_copy(outv, out_h)


_sc_kernel = functools.partial(
    pl.kernel,
    mesh=plsc.VectorSubcoreMesh(core_axis_name="c", subcore_axis_name="s",
                                num_cores=1),
    compiler_params=pltpu.CompilerParams(needs_layout_passes=False),
    out_type=(jax.ShapeDtypeStruct((_MAX_BOX, _L), jnp.float32),
              jax.ShapeDtypeStruct((2, _NS, _L), jnp.float32)),
    scratch_types=[
        pltpu.VMEM((_PER,), jnp.float32),
        pltpu.VMEM((_PER,), jnp.float32),
        pltpu.VMEM((_PER,), jnp.float32),
        pltpu.VMEM((_PER,), jnp.float32),
        pltpu.VMEM((_PER,), jnp.float32),
        pltpu.VMEM((_PER,), jnp.float32),
        pltpu.VMEM((_L,), jnp.float32),
        pltpu.VMEM((_NS, _L), jnp.float32),
        pltpu.VMEM((_MAX_BOX, _L), jnp.float32),
    ],
)(_sc_body)


def kernel(boxes, scores):
    n = scores.shape[0]
    boxes = boxes.astype(jnp.float32)
    pad = _NPAD - n
    x1 = jnp.pad(boxes[:, 0], (0, pad))
    y1 = jnp.pad(boxes[:, 1], (0, pad))
    x2 = jnp.pad(boxes[:, 2], (0, pad))
    y2 = jnp.pad(boxes[:, 3], (0, pad))
    sc = jnp.pad(scores.astype(jnp.float32), (0, pad), constant_values=_NEG)
    out, _ = _sc_kernel(x1, y1, x2, y2, sc)
    s = out[:, 0]
    valid = (s >= _SCORE_THRESH).astype(boxes.dtype)
    kept_boxes = out[:, 2:6] * valid[:, None]
    kept_scores = s * valid
    return jnp.concatenate([kept_boxes, kept_scores[:, None]], axis=1)
